# baseline (speedup 1.0000x reference)
import functools

import jax
import jax.numpy as jnp
from jax import lax
from jax.experimental import pallas as pl
from jax.experimental.pallas import tpu as pltpu
from jax.experimental.pallas import tpu_sc as plsc

N = 10000
E = 320000
D = 128
NUM_GRAPHS = 16

NC = 2
NS = 16
NW = NC * NS
EDGES_PW = E // NW
DCHUNK = 80
DNCHUNK = EDGES_PW // DCHUNK
CHUNK = 40
NCHUNK = EDGES_PW // CHUNK
ROWS_PT = 624
REM_OFF = ROWS_PT * NS
REM = N - REM_OFF


def _tiled_copy(src, dst, s):
    pltpu.sync_copy(src.at[pl.ds(s * ROWS_PT, ROWS_PT)],
                    dst.at[pl.ds(s * ROWS_PT, ROWS_PT)])

    @pl.when(s == NS - 1)
    def _():
        pltpu.sync_copy(src.at[pl.ds(REM_OFF, REM)],
                        dst.at[pl.ds(REM_OFF, REM)])

_MESH = plsc.VectorSubcoreMesh(core_axis_name="c", subcore_axis_name="s")


N_PAD = 10240
PT_1D = N_PAD // NS


DNBUF = 5
DROUNDS = DNCHUNK // DNBUF


@functools.partial(
    pl.kernel,
    mesh=_MESH,
    out_type=jax.ShapeDtypeStruct((NC, N_PAD), jnp.float32),
    scratch_types=[
        pltpu.VMEM((DNBUF, DCHUNK), jnp.int32),
        pltpu.VMEM((DCHUNK,), jnp.float32),
        pltpu.SemaphoreType.DMA((DNBUF,)),
        pltpu.SemaphoreType.DMA((DNBUF,)),
        pltpu.VMEM_SHARED((N_PAD,), jnp.float32),
    ],
)
def _deg_kernel(dst_hbm, ones_hbm, out_hbm, dst_v, ones_v, d_sems, s_sems,
                deg_sh):
    c = lax.axis_index("c")
    s = lax.axis_index("s")
    wid = s * NC + c
    pltpu.sync_copy(ones_hbm.at[pl.ds(s * PT_1D, PT_1D)],
                    deg_sh.at[pl.ds(s * PT_1D, PT_1D)])
    pltpu.sync_copy(ones_hbm.at[pl.ds(0, DCHUNK)], ones_v)
    plsc.subcore_barrier()
    base0 = wid * EDGES_PW

    def start_idx(i, j):
        pltpu.async_copy(dst_hbm.at[pl.ds(base0 + i * DCHUNK, DCHUNK)],
                         dst_v.at[j], d_sems.at[j])

    def do_round(k, reissue):
        for j in range(DNBUF):
            pltpu.make_async_copy(dst_hbm.at[pl.ds(0, DCHUNK)],
                                  dst_v.at[j], d_sems.at[j]).wait()
            pltpu.async_copy(ones_v, deg_sh.at[dst_v.at[j]], s_sems.at[j],
                             add=True)
        for j in range(DNBUF):
            pltpu.make_async_copy(ones_v, deg_sh.at[dst_v.at[j]],
                                  s_sems.at[j]).wait()
            if reissue:
                start_idx(k * DNBUF + j + DNBUF, j)

    for j in range(DNBUF):
        start_idx(j, j)
    lax.fori_loop(0, DROUNDS - 1,
                  lambda k, carry: (do_round(k, True), carry)[1], 0)
    do_round(DROUNDS - 1, False)
    plsc.subcore_barrier()
    pltpu.sync_copy(deg_sh.at[pl.ds(s * PT_1D, PT_1D)],
                    out_hbm.at[c].at[pl.ds(s * PT_1D, PT_1D)])


NBUF = 5
ROUNDS = NCHUNK // NBUF


@functools.partial(
    pl.kernel,
    mesh=_MESH,
    out_type=jax.ShapeDtypeStruct((NC, N, D), jnp.float32),
    scratch_types=[
        pltpu.VMEM((EDGES_PW,), jnp.int32),
        pltpu.VMEM((NBUF, CHUNK), jnp.int32),
        pltpu.VMEM((NBUF, CHUNK, D), jnp.float32),
        pltpu.SemaphoreType.DMA((NBUF,)),
        pltpu.SemaphoreType.DMA((NBUF,)),
        pltpu.SemaphoreType.DMA((NBUF,)),
        pltpu.VMEM_SHARED((N, D), jnp.float32),
    ],
)
def _edge_kernel(g_hbm, src_hbm, dst_hbm, out_hbm,
                 src_all, dst_v, rows_v, g_sems, d_sems, s_sems, acc_sh):
    c = lax.axis_index("c")
    s = lax.axis_index("s")
    wid = s * NC + c
    base0 = wid * EDGES_PW
    pltpu.sync_copy(src_hbm.at[pl.ds(base0, EDGES_PW)], src_all)
    _tiled_copy(g_hbm, acc_sh, s)
    plsc.subcore_barrier()

    def start_fetch(i, j):
        pltpu.async_copy(dst_hbm.at[pl.ds(base0 + i * CHUNK, CHUNK)],
                         dst_v.at[j], d_sems.at[j])
        pltpu.async_copy(g_hbm.at[src_all.at[pl.ds(i * CHUNK, CHUNK)]],
                         rows_v.at[j], g_sems.at[j])

    def wait_fetch(j):
        pltpu.make_async_copy(dst_hbm.at[pl.ds(0, CHUNK)],
                              dst_v.at[j], d_sems.at[j]).wait()
        pltpu.make_async_copy(g_hbm.at[src_all.at[pl.ds(0, CHUNK)]],
                              rows_v.at[j], g_sems.at[j]).wait()

    def start_scatter(j):
        pltpu.async_copy(rows_v.at[j], acc_sh.at[dst_v.at[j]], s_sems.at[j],
                         add=True)

    def wait_scatter(j):
        pltpu.make_async_copy(rows_v.at[j], acc_sh.at[dst_v.at[j]],
                              s_sems.at[j]).wait()

    for j in range(NBUF):
        start_fetch(j, j)

    def do_round(k, reissue):
        for j in range(NBUF):
            wait_fetch(j)
            start_scatter(j)
        for j in range(NBUF):
            wait_scatter(j)
            if reissue:
                start_fetch(k * NBUF + j + NBUF, j)

    lax.fori_loop(0, ROUNDS - 1,
                  lambda k, carry: (do_round(k, True), carry)[1], 0)
    do_round(ROUNDS - 1, False)
    plsc.subcore_barrier()
    _tiled_copy(acc_sh, out_hbm.at[c], s)


NB = 10
BLK = N // NB


def _k1_body(x_ref, W_ref, deg_ref, dinv_ref, g_ref):
    deg = deg_ref[0] + deg_ref[1] - 1.0
    dinv = lax.rsqrt(deg)
    dinv_ref[...] = dinv
    g_ref[...] = jnp.dot(x_ref[...], W_ref[...],
                         preferred_element_type=jnp.float32) * dinv


def _k1(x, W1, deg_pair):
    return pl.pallas_call(
        _k1_body,
        grid=(NB,),
        in_specs=[
            pl.BlockSpec((BLK, D), lambda i: (i, 0)),
            pl.BlockSpec((D, D), lambda i: (0, 0)),
            pl.BlockSpec((NC, BLK, 1), lambda i: (0, i, 0)),
        ],
        out_specs=[
            pl.BlockSpec((BLK, 1), lambda i: (i, 0)),
            pl.BlockSpec((BLK, D), lambda i: (i, 0)),
        ],
        out_shape=[
            jax.ShapeDtypeStruct((N, 1), jnp.float32),
            jax.ShapeDtypeStruct((N, D), jnp.float32),
        ],
    )(x, W1, deg_pair)


def _mid_body(acc_ref, g_ref, dinv_ref, b_ref, W_ref, h_ref, gn_ref):
    dinv = dinv_ref[...]
    h = (acc_ref[0] + acc_ref[1] - g_ref[...]) * dinv + b_ref[...]
    h_ref[...] = h
    gn_ref[...] = jnp.dot(h, W_ref[...],
                          preferred_element_type=jnp.float32) * dinv


def _mid(acc, g, dinv, b, Wn):
    return pl.pallas_call(
        _mid_body,
        grid=(NB,),
        in_specs=[
            pl.BlockSpec((NC, BLK, D), lambda i: (0, i, 0)),
            pl.BlockSpec((BLK, D), lambda i: (i, 0)),
            pl.BlockSpec((BLK, 1), lambda i: (i, 0)),
            pl.BlockSpec((1, D), lambda i: (0, 0)),
            pl.BlockSpec((D, D), lambda i: (0, 0)),
        ],
        out_specs=[
            pl.BlockSpec((BLK, D), lambda i: (i, 0)),
            pl.BlockSpec((BLK, D), lambda i: (i, 0)),
        ],
        out_shape=[
            jax.ShapeDtypeStruct((N, D), jnp.float32),
            jax.ShapeDtypeStruct((N, D), jnp.float32),
        ],
    )(acc, g, dinv, b.reshape(1, D), Wn)


def _k4_body(acc_ref, g_ref, dinv_ref, b_ref, h1_ref, h2_ref, batch_ref,
             Wr_ref, br_ref, Wf_ref, bf_ref, Wo_ref, bo_ref,
             out_ref, sums_sc, cnt_sc):
    i = pl.program_id(0)
    h3 = (acc_ref[0] + acc_ref[1] - g_ref[...]) * dinv_ref[...] + b_ref[...]
    oh = (batch_ref[...] ==
          lax.broadcasted_iota(jnp.int32, (BLK, NUM_GRAPHS), 1)
          ).astype(jnp.float32)
    dn = (((0,), (0,)), ((), ()))
    s1 = lax.dot_general(oh, h1_ref[...], dn, preferred_element_type=jnp.float32)
    s2 = lax.dot_general(oh, h2_ref[...], dn, preferred_element_type=jnp.float32)
    s3 = lax.dot_general(oh, h3, dn, preferred_element_type=jnp.float32)
    cnt = jnp.sum(oh, axis=0).reshape(NUM_GRAPHS, 1)

    @pl.when(i == 0)
    def _():
        sums_sc[...] = jnp.zeros_like(sums_sc)
        cnt_sc[...] = jnp.zeros_like(cnt_sc)

    sums_sc[:, 0:D] += s1
    sums_sc[:, D:2 * D] += s2
    sums_sc[:, 2 * D:3 * D] += s3
    cnt_sc[...] += cnt

    @pl.when(i == NB - 1)
    def _():
        pooled = sums_sc[...] / jnp.maximum(cnt_sc[...], 1.0)
        r = jnp.maximum(
            jnp.dot(pooled, Wr_ref[...], preferred_element_type=jnp.float32)
            + br_ref[...], 0.0)
        f = jnp.maximum(
            jnp.dot(r, Wf_ref[...], preferred_element_type=jnp.float32)
            + bf_ref[...], 0.0)
        out_ref[...] = (jnp.dot(f, Wo_ref[...], preferred_element_type=jnp.float32)
                        + bo_ref[...])


def _k4(acc, g3, dinv, b3, h1, h2, batch2d, Wr, br, Wf, bf, Wo, bo):
    fr = Wr.shape[0]
    fm = Wf.shape[0]
    fo = Wo.shape[0]
    od = Wo.shape[1]
    return pl.pallas_call(
        _k4_body,
        grid=(NB,),
        in_specs=[
            pl.BlockSpec((NC, BLK, D), lambda i: (0, i, 0)),
            pl.BlockSpec((BLK, D), lambda i: (i, 0)),
            pl.BlockSpec((BLK, 1), lambda i: (i, 0)),
            pl.BlockSpec((1, D), lambda i: (0, 0)),
            pl.BlockSpec((BLK, D), lambda i: (i, 0)),
            pl.BlockSpec((BLK, D), lambda i: (i, 0)),
            pl.BlockSpec((BLK, 1), lambda i: (i, 0)),
            pl.BlockSpec((fr, fm), lambda i: (0, 0)),
            pl.BlockSpec((1, fm), lambda i: (0, 0)),
            pl.BlockSpec((fm, fo), lambda i: (0, 0)),
            pl.BlockSpec((1, fo), lambda i: (0, 0)),
            pl.BlockSpec((fo, od), lambda i: (0, 0)),
            pl.BlockSpec((1, od), lambda i: (0, 0)),
        ],
        out_specs=pl.BlockSpec((NUM_GRAPHS, od), lambda i: (0, 0)),
        out_shape=jax.ShapeDtypeStruct((NUM_GRAPHS, od), jnp.float32),
        scratch_shapes=[
            pltpu.VMEM((NUM_GRAPHS, 3 * D), jnp.float32),
            pltpu.VMEM((NUM_GRAPHS, 1), jnp.float32),
        ],
    )(acc, g3, dinv, b3.reshape(1, D), h1, h2, batch2d,
      Wr, br.reshape(1, fm), Wf, bf.reshape(1, fo), Wo, bo.reshape(1, od))


def kernel(x, edge_index, batch, W1, b1, W2, b2, W3, b3, Wr, br, Wf, bf, Wo, bo):
    src = edge_index[0]
    dst = edge_index[1]
    ones_n = jnp.ones((N_PAD,), jnp.float32)
    deg_pair = _deg_kernel(dst, ones_n)
    dinv, g1 = _k1(x, W1, deg_pair.reshape(NC, N_PAD, 1))
    acc1 = _edge_kernel(g1, src, dst)
    h1, g2 = _mid(acc1, g1, dinv, b1, W2)
    acc2 = _edge_kernel(g2, src, dst)
    h2, g3 = _mid(acc2, g2, dinv, b2, W3)
    acc3 = _edge_kernel(g3, src, dst)
    return _k4(acc3, g3, dinv, b3, h1, h2, batch.reshape(N, 1),
               Wr, br, Wf, bf, Wo, bo)


# speedup vs baseline: 1.1564x; 1.1564x over previous
import functools

import jax
import jax.numpy as jnp
from jax import lax
from jax.experimental import pallas as pl
from jax.experimental.pallas import tpu as pltpu
from jax.experimental.pallas import tpu_sc as plsc

N = 10000
E = 320000
D = 128
NUM_GRAPHS = 16

NC = 2
NS = 16
NW = NC * NS
EDGES_PW = E // NW
DCHUNK = 80
DNCHUNK = EDGES_PW // DCHUNK
CHUNK = 40
NCHUNK = EDGES_PW // CHUNK
ROWS_PT = 624
REM_OFF = ROWS_PT * NS
REM = N - REM_OFF


def _tiled_copy(src, dst, s):
    pltpu.sync_copy(src.at[pl.ds(s * ROWS_PT, ROWS_PT)],
                    dst.at[pl.ds(s * ROWS_PT, ROWS_PT)])

    @pl.when(s == NS - 1)
    def _():
        pltpu.sync_copy(src.at[pl.ds(REM_OFF, REM)],
                        dst.at[pl.ds(REM_OFF, REM)])

_MESH = plsc.VectorSubcoreMesh(core_axis_name="c", subcore_axis_name="s")


N_PAD = 10240
PT_1D = N_PAD // NS


DNBUF = 5
DROUNDS = DNCHUNK // DNBUF


@functools.partial(
    pl.kernel,
    mesh=_MESH,
    out_type=jax.ShapeDtypeStruct((NC, N_PAD), jnp.float32),
    scratch_types=[
        pltpu.VMEM((DNBUF, DCHUNK), jnp.int32),
        pltpu.VMEM((DCHUNK,), jnp.float32),
        pltpu.SemaphoreType.DMA((DNBUF,)),
        pltpu.SemaphoreType.DMA((DNBUF,)),
        pltpu.VMEM_SHARED((N_PAD,), jnp.float32),
    ],
)
def _deg_kernel(dst_hbm, ones_hbm, out_hbm, dst_v, ones_v, d_sems, s_sems,
                deg_sh):
    c = lax.axis_index("c")
    s = lax.axis_index("s")
    wid = s * NC + c
    pltpu.sync_copy(ones_hbm.at[pl.ds(s * PT_1D, PT_1D)],
                    deg_sh.at[pl.ds(s * PT_1D, PT_1D)])
    pltpu.sync_copy(ones_hbm.at[pl.ds(0, DCHUNK)], ones_v)
    plsc.subcore_barrier()
    base0 = wid * EDGES_PW

    def start_idx(i, j):
        pltpu.async_copy(dst_hbm.at[pl.ds(base0 + i * DCHUNK, DCHUNK)],
                         dst_v.at[j], d_sems.at[j])

    def do_round(k, reissue):
        for j in range(DNBUF):
            pltpu.make_async_copy(dst_hbm.at[pl.ds(0, DCHUNK)],
                                  dst_v.at[j], d_sems.at[j]).wait()
            pltpu.async_copy(ones_v, deg_sh.at[dst_v.at[j]], s_sems.at[j],
                             add=True)
        for j in range(DNBUF):
            pltpu.make_async_copy(ones_v, deg_sh.at[dst_v.at[j]],
                                  s_sems.at[j]).wait()
            if reissue:
                start_idx(k * DNBUF + j + DNBUF, j)

    for j in range(DNBUF):
        start_idx(j, j)
    lax.fori_loop(0, DROUNDS - 1,
                  lambda k, carry: (do_round(k, True), carry)[1], 0)
    do_round(DROUNDS - 1, False)
    plsc.subcore_barrier()
    pltpu.sync_copy(deg_sh.at[pl.ds(s * PT_1D, PT_1D)],
                    out_hbm.at[c].at[pl.ds(s * PT_1D, PT_1D)])


NBUF = 5
ROUNDS = NCHUNK // NBUF


@functools.partial(
    pl.kernel,
    mesh=_MESH,
    out_type=jax.ShapeDtypeStruct((NC, N, D), jnp.float32),
    scratch_types=[
        pltpu.VMEM((EDGES_PW,), jnp.int32),
        pltpu.VMEM((NBUF, CHUNK), jnp.int32),
        pltpu.VMEM((NBUF, CHUNK, D), jnp.float32),
        pltpu.SemaphoreType.DMA((NBUF,)),
        pltpu.SemaphoreType.DMA((NBUF,)),
        pltpu.SemaphoreType.DMA((NBUF,)),
        pltpu.VMEM_SHARED((N, D), jnp.float32),
    ],
)
def _edge_kernel(g_hbm, src_hbm, dst_hbm, out_hbm,
                 src_all, dst_v, rows_v, g_sems, d_sems, s_sems, acc_sh):
    c = lax.axis_index("c")
    s = lax.axis_index("s")
    wid = s * NC + c
    base0 = wid * EDGES_PW
    pltpu.sync_copy(src_hbm.at[pl.ds(base0, EDGES_PW)], src_all)
    _tiled_copy(g_hbm, acc_sh, s)
    plsc.subcore_barrier()

    def start_fetch(i, j):
        pltpu.async_copy(dst_hbm.at[pl.ds(base0 + i * CHUNK, CHUNK)],
                         dst_v.at[j], d_sems.at[j])
        pltpu.async_copy(g_hbm.at[src_all.at[pl.ds(i * CHUNK, CHUNK)]],
                         rows_v.at[j], g_sems.at[j])

    def wait_fetch(j):
        pltpu.make_async_copy(dst_hbm.at[pl.ds(0, CHUNK)],
                              dst_v.at[j], d_sems.at[j]).wait()
        pltpu.make_async_copy(g_hbm.at[src_all.at[pl.ds(0, CHUNK)]],
                              rows_v.at[j], g_sems.at[j]).wait()

    def start_scatter(j):
        pltpu.async_copy(rows_v.at[j], acc_sh.at[dst_v.at[j]], s_sems.at[j],
                         add=True)

    def wait_scatter(j):
        pltpu.make_async_copy(rows_v.at[j], acc_sh.at[dst_v.at[j]],
                              s_sems.at[j]).wait()

    for j in range(NBUF):
        start_fetch(j, j)

    def do_round(k, reissue):
        for j in range(NBUF):
            wait_fetch(j)
            start_scatter(j)
            wait_scatter(j)
            if reissue:
                start_fetch(k * NBUF + j + NBUF, j)

    lax.fori_loop(0, ROUNDS - 1,
                  lambda k, carry: (do_round(k, True), carry)[1], 0)
    do_round(ROUNDS - 1, False)
    plsc.subcore_barrier()
    _tiled_copy(acc_sh, out_hbm.at[c], s)


NB = 10
BLK = N // NB


def _k1_body(x_ref, W_ref, deg_ref, dinv_ref, g_ref):
    deg = deg_ref[0] + deg_ref[1] - 1.0
    dinv = lax.rsqrt(deg)
    dinv_ref[...] = dinv
    g_ref[...] = jnp.dot(x_ref[...], W_ref[...],
                         preferred_element_type=jnp.float32) * dinv


def _k1(x, W1, deg_pair):
    return pl.pallas_call(
        _k1_body,
        grid=(NB,),
        in_specs=[
            pl.BlockSpec((BLK, D), lambda i: (i, 0)),
            pl.BlockSpec((D, D), lambda i: (0, 0)),
            pl.BlockSpec((NC, BLK, 1), lambda i: (0, i, 0)),
        ],
        out_specs=[
            pl.BlockSpec((BLK, 1), lambda i: (i, 0)),
            pl.BlockSpec((BLK, D), lambda i: (i, 0)),
        ],
        out_shape=[
            jax.ShapeDtypeStruct((N, 1), jnp.float32),
            jax.ShapeDtypeStruct((N, D), jnp.float32),
        ],
    )(x, W1, deg_pair)


def _mid_body(acc_ref, g_ref, dinv_ref, b_ref, W_ref, h_ref, gn_ref):
    dinv = dinv_ref[...]
    h = (acc_ref[0] + acc_ref[1] - g_ref[...]) * dinv + b_ref[...]
    h_ref[...] = h
    gn_ref[...] = jnp.dot(h, W_ref[...],
                          preferred_element_type=jnp.float32) * dinv


def _mid(acc, g, dinv, b, Wn):
    return pl.pallas_call(
        _mid_body,
        grid=(NB,),
        in_specs=[
            pl.BlockSpec((NC, BLK, D), lambda i: (0, i, 0)),
            pl.BlockSpec((BLK, D), lambda i: (i, 0)),
            pl.BlockSpec((BLK, 1), lambda i: (i, 0)),
            pl.BlockSpec((1, D), lambda i: (0, 0)),
            pl.BlockSpec((D, D), lambda i: (0, 0)),
        ],
        out_specs=[
            pl.BlockSpec((BLK, D), lambda i: (i, 0)),
            pl.BlockSpec((BLK, D), lambda i: (i, 0)),
        ],
        out_shape=[
            jax.ShapeDtypeStruct((N, D), jnp.float32),
            jax.ShapeDtypeStruct((N, D), jnp.float32),
        ],
    )(acc, g, dinv, b.reshape(1, D), Wn)


def _k4_body(acc_ref, g_ref, dinv_ref, b_ref, h1_ref, h2_ref, batch_ref,
             Wr_ref, br_ref, Wf_ref, bf_ref, Wo_ref, bo_ref,
             out_ref, sums_sc, cnt_sc):
    i = pl.program_id(0)
    h3 = (acc_ref[0] + acc_ref[1] - g_ref[...]) * dinv_ref[...] + b_ref[...]
    oh = (batch_ref[...] ==
          lax.broadcasted_iota(jnp.int32, (BLK, NUM_GRAPHS), 1)
          ).astype(jnp.float32)
    dn = (((0,), (0,)), ((), ()))
    s1 = lax.dot_general(oh, h1_ref[...], dn, preferred_element_type=jnp.float32)
    s2 = lax.dot_general(oh, h2_ref[...], dn, preferred_element_type=jnp.float32)
    s3 = lax.dot_general(oh, h3, dn, preferred_element_type=jnp.float32)
    cnt = jnp.sum(oh, axis=0).reshape(NUM_GRAPHS, 1)

    @pl.when(i == 0)
    def _():
        sums_sc[...] = jnp.zeros_like(sums_sc)
        cnt_sc[...] = jnp.zeros_like(cnt_sc)

    sums_sc[:, 0:D] += s1
    sums_sc[:, D:2 * D] += s2
    sums_sc[:, 2 * D:3 * D] += s3
    cnt_sc[...] += cnt

    @pl.when(i == NB - 1)
    def _():
        pooled = sums_sc[...] / jnp.maximum(cnt_sc[...], 1.0)
        r = jnp.maximum(
            jnp.dot(pooled, Wr_ref[...], preferred_element_type=jnp.float32)
            + br_ref[...], 0.0)
        f = jnp.maximum(
            jnp.dot(r, Wf_ref[...], preferred_element_type=jnp.float32)
            + bf_ref[...], 0.0)
        out_ref[...] = (jnp.dot(f, Wo_ref[...], preferred_element_type=jnp.float32)
                        + bo_ref[...])


def _k4(acc, g3, dinv, b3, h1, h2, batch2d, Wr, br, Wf, bf, Wo, bo):
    fr = Wr.shape[0]
    fm = Wf.shape[0]
    fo = Wo.shape[0]
    od = Wo.shape[1]
    return pl.pallas_call(
        _k4_body,
        grid=(NB,),
        in_specs=[
            pl.BlockSpec((NC, BLK, D), lambda i: (0, i, 0)),
            pl.BlockSpec((BLK, D), lambda i: (i, 0)),
            pl.BlockSpec((BLK, 1), lambda i: (i, 0)),
            pl.BlockSpec((1, D), lambda i: (0, 0)),
            pl.BlockSpec((BLK, D), lambda i: (i, 0)),
            pl.BlockSpec((BLK, D), lambda i: (i, 0)),
            pl.BlockSpec((BLK, 1), lambda i: (i, 0)),
            pl.BlockSpec((fr, fm), lambda i: (0, 0)),
            pl.BlockSpec((1, fm), lambda i: (0, 0)),
            pl.BlockSpec((fm, fo), lambda i: (0, 0)),
            pl.BlockSpec((1, fo), lambda i: (0, 0)),
            pl.BlockSpec((fo, od), lambda i: (0, 0)),
            pl.BlockSpec((1, od), lambda i: (0, 0)),
        ],
        out_specs=pl.BlockSpec((NUM_GRAPHS, od), lambda i: (0, 0)),
        out_shape=jax.ShapeDtypeStruct((NUM_GRAPHS, od), jnp.float32),
        scratch_shapes=[
            pltpu.VMEM((NUM_GRAPHS, 3 * D), jnp.float32),
            pltpu.VMEM((NUM_GRAPHS, 1), jnp.float32),
        ],
    )(acc, g3, dinv, b3.reshape(1, D), h1, h2, batch2d,
      Wr, br.reshape(1, fm), Wf, bf.reshape(1, fo), Wo, bo.reshape(1, od))


def kernel(x, edge_index, batch, W1, b1, W2, b2, W3, b3, Wr, br, Wf, bf, Wo, bo):
    src = edge_index[0]
    dst = edge_index[1]
    ones_n = jnp.ones((N_PAD,), jnp.float32)
    deg_pair = _deg_kernel(dst, ones_n)
    dinv, g1 = _k1(x, W1, deg_pair.reshape(NC, N_PAD, 1))
    acc1 = _edge_kernel(g1, src, dst)
    h1, g2 = _mid(acc1, g1, dinv, b1, W2)
    acc2 = _edge_kernel(g2, src, dst)
    h2, g3 = _mid(acc2, g2, dinv, b2, W3)
    acc3 = _edge_kernel(g3, src, dst)
    return _k4(acc3, g3, dinv, b3, h1, h2, batch.reshape(N, 1),
               Wr, br, Wf, bf, Wo, bo)
